# baseline (speedup 1.0000x reference)
import jax
import jax.numpy as jnp
from jax.experimental import pallas as pl
from jax.experimental.pallas import tpu as pltpu

_G = 2
_NSL = 2
_NST = 2
_H = 1024


def _stream_kernel(mask_ref, w_ref, b_ref, emb_ref, emb_out_ref, logits_ref,
                   buf, lgbuf, sem_in, sem_out, sem_lg):
    n_tiles, S, D = emb_ref.shape
    L = w_ref.shape[0]
    n_chunks = n_tiles // _G

    def in_copies(i, slot):
        cs = []
        for g in range(_G):
            for j in range(_NST):
                cs.append(pltpu.make_async_copy(
                    emb_ref.at[i * _G + g, pl.ds(j * _H, _H), :],
                    buf.at[slot, g, pl.ds(j * _H, _H), :],
                    sem_in.at[slot, g, j]))
        return cs

    def out_copies(i, slot):
        cs = []
        for g in range(_G):
            for j in range(_NST):
                cs.append(pltpu.make_async_copy(
                    buf.at[slot, g, pl.ds(j * _H, _H), :],
                    emb_out_ref.at[i * _G + g, pl.ds(j * _H, _H), :],
                    sem_out.at[slot, g, j]))
        return cs

    def lg_copy(i, slot):
        return pltpu.make_async_copy(
            lgbuf.at[slot], logits_ref.at[pl.ds(i * _G, _G)], sem_lg.at[slot])

    for c in in_copies(0, 0):
        c.start()

    def body(i, _):
        slot = jax.lax.rem(i, _NSL)
        for c in in_copies(i, slot):
            c.wait()
        for c in out_copies(i, slot):
            c.start()

        @pl.when(i >= _NSL)
        def _():
            lg_copy(i - _NSL, slot).wait()

        for g in range(_G):
            tile = i * _G + g
            lyr = jax.lax.rem(tile, L)
            bidx = jax.lax.div(tile, L)
            y = jnp.dot(buf[slot, g], w_ref[lyr],
                        preferred_element_type=jnp.float32)
            lgbuf[slot, g] = y + b_ref[lyr] + mask_ref[bidx]
        lg_copy(i, slot).start()

        for c in out_copies(i, slot):
            c.wait()

        @pl.when(i + 1 < n_chunks)
        def _():
            for c in in_copies(i + 1, jax.lax.rem(i + 1, _NSL)):
                c.start()
        return 0

    jax.lax.fori_loop(0, n_chunks, body, 0)

    for c in range(max(0, n_chunks - _NSL), n_chunks):
        lg_copy(c, c % _NSL).wait()


@jax.jit
def _run(emb_flat, mask, W, b3):
    T, S, D = emb_flat.shape
    L, _, C = W.shape

    emb_out, logits = pl.pallas_call(
        _stream_kernel,
        in_specs=[
            pl.BlockSpec(memory_space=pltpu.MemorySpace.VMEM),
            pl.BlockSpec(memory_space=pltpu.MemorySpace.VMEM),
            pl.BlockSpec(memory_space=pltpu.MemorySpace.VMEM),
            pl.BlockSpec(memory_space=pltpu.MemorySpace.HBM),
        ],
        out_specs=[
            pl.BlockSpec(memory_space=pltpu.MemorySpace.HBM),
            pl.BlockSpec(memory_space=pltpu.MemorySpace.HBM),
        ],
        out_shape=[
            jax.ShapeDtypeStruct((T, S, D), jnp.float32),
            jax.ShapeDtypeStruct((T, S, C), jnp.float32),
        ],
        scratch_shapes=[
            pltpu.VMEM((_NSL, _G, S, D), jnp.float32),
            pltpu.VMEM((_NSL, _G, S, C), jnp.float32),
            pltpu.SemaphoreType.DMA((_NSL, _G, _NST)),
            pltpu.SemaphoreType.DMA((_NSL, _G, _NST)),
            pltpu.SemaphoreType.DMA((_NSL,)),
        ],
    )(mask, W, b3, emb_flat)
    return emb_out, logits


def kernel(emb_sentences, att_sentences, W, b):
    B, L, S, D = emb_sentences.shape
    C = W.shape[-1]
    mask = jnp.where(att_sentences, 0.0, -jnp.inf).astype(jnp.float32)
    mask = mask.reshape(B, S, 1)
    b3 = b.reshape(L, 1, C)
    emb_flat = emb_sentences.reshape(B * L, S, D)
    emb_out, logits = _run(emb_flat, mask, W, b3)
    return (emb_out.reshape(B, L, S, D), att_sentences,
            logits.reshape(B, L, S, C))


# speedup vs baseline: 1.1228x; 1.1228x over previous
import jax
import jax.numpy as jnp
from jax.experimental import pallas as pl
from jax.experimental.pallas import tpu as pltpu

_NBUF = 6
_K = 3
_NLG = 2


def _stream_kernel(mask_ref, w_ref, b_ref, emb_ref, emb_out_ref, logits_ref,
                   buf, lgbuf, sem_in, sem_out, sem_lg):
    n_tiles, S, D = emb_ref.shape
    L = w_ref.shape[0]

    def in_copy(i, slot):
        return pltpu.make_async_copy(emb_ref.at[i], buf.at[slot], sem_in.at[slot])

    def out_copy(i, slot):
        return pltpu.make_async_copy(buf.at[slot], emb_out_ref.at[i], sem_out.at[slot])

    def lg_copy(i, slot):
        return pltpu.make_async_copy(lgbuf.at[slot], logits_ref.at[i], sem_lg.at[slot])

    for j in range(_K):
        in_copy(j, j).start()

    def body(i, _):
        slot = jax.lax.rem(i, _NBUF)
        in_copy(i, slot).wait()
        out_copy(i, slot).start()

        @pl.when(i + _K < n_tiles)
        def _():
            nxt = i + _K
            slot2 = jax.lax.rem(nxt, _NBUF)

            @pl.when(nxt >= _NBUF)
            def _():
                out_copy(nxt - _NBUF, slot2).wait()

            in_copy(nxt, slot2).start()

        lyr = jax.lax.rem(i, L)
        bidx = jax.lax.div(i, L)
        lslot = jax.lax.rem(i, _NLG)

        @pl.when(i >= _NLG)
        def _():
            lg_copy(i - _NLG, lslot).wait()

        y = jnp.dot(buf[slot], w_ref[lyr], preferred_element_type=jnp.float32)
        lgbuf[lslot] = y + b_ref[lyr] + mask_ref[bidx]
        lg_copy(i, lslot).start()
        return 0

    jax.lax.fori_loop(0, n_tiles, body, 0)

    for c in range(max(0, n_tiles - _NBUF), n_tiles):
        out_copy(c, c % _NBUF).wait()
    for c in range(max(0, n_tiles - _NLG), n_tiles):
        lg_copy(c, c % _NLG).wait()


@jax.jit
def _run(emb_flat, mask, W, b3):
    T, S, D = emb_flat.shape
    L, _, C = W.shape

    emb_out, logits = pl.pallas_call(
        _stream_kernel,
        in_specs=[
            pl.BlockSpec(memory_space=pltpu.MemorySpace.VMEM),
            pl.BlockSpec(memory_space=pltpu.MemorySpace.VMEM),
            pl.BlockSpec(memory_space=pltpu.MemorySpace.VMEM),
            pl.BlockSpec(memory_space=pltpu.MemorySpace.HBM),
        ],
        out_specs=[
            pl.BlockSpec(memory_space=pltpu.MemorySpace.HBM),
            pl.BlockSpec(memory_space=pltpu.MemorySpace.HBM),
        ],
        out_shape=[
            jax.ShapeDtypeStruct((T, S, D), jnp.float32),
            jax.ShapeDtypeStruct((T, S, C), jnp.float32),
        ],
        scratch_shapes=[
            pltpu.VMEM((_NBUF, S, D), jnp.float32),
            pltpu.VMEM((_NLG, S, C), jnp.float32),
            pltpu.SemaphoreType.DMA((_NBUF,)),
            pltpu.SemaphoreType.DMA((_NBUF,)),
            pltpu.SemaphoreType.DMA((_NLG,)),
        ],
    )(mask, W, b3, emb_flat)
    return emb_out, logits


def kernel(emb_sentences, att_sentences, W, b):
    B, L, S, D = emb_sentences.shape
    C = W.shape[-1]
    mask = jnp.where(att_sentences, 0.0, -jnp.inf).astype(jnp.float32)
    mask = mask.reshape(B, S, 1)
    b3 = b.reshape(L, 1, C)
    emb_flat = emb_sentences.reshape(B * L, S, D)
    emb_out, logits = _run(emb_flat, mask, W, b3)
    return (emb_out.reshape(B, L, S, D), att_sentences,
            logits.reshape(B, L, S, C))
